# initial kernel scaffold (unmeasured)
import jax
import jax.numpy as jnp
from jax import lax
from jax.experimental import pallas as pl
from jax.experimental.pallas import tpu as pltpu

SQ = 1024
SKV_SHARD = 1024
HQ = 8
DH = 128
D = HQ * DH
WIN = 128
SCALE = 0.08838834764831843

QBLK = 256
NBLK = SQ // QBLK
KBAND = 512
SLIVER = 128
KV_USED = SKV_SHARD + SLIVER

_KSTART = (0, 128, 384, 640)


def _mm(a, b):
    return lax.dot_general(a, b, (((1,), (0,)), ((), ())),
                           preferred_element_type=jnp.float32)


def kernel(x, Wq, K_ext, V_ext, Wo):
    x2 = x.reshape(SQ, D)
    k3 = K_ext.reshape(SKV_SHARD, HQ, DH)
    v3 = V_ext.reshape(SKV_SHARD, HQ, DH)

    def body(x_ref, wq_ref, k_ref, v_ref, wo_ref, out_ref,
             qbuf, kbuf, vbuf, ctxbuf,
             sliver_send_sems, sliver_recv_sems,
             send_sems, recv_sems, relay_sems):
        my = lax.axis_index("i")

        def blk(ref, b):
            return ref.at[pl.ds(b * QBLK, QBLK), :]

        @pl.when(my == 1)
        def _():
            sends = []
            for i, (src, dst) in enumerate(((k_ref, kbuf), (v_ref, vbuf))):
                rd = pltpu.make_async_remote_copy(
                    src_ref=src.at[pl.ds(0, SLIVER)],
                    dst_ref=dst.at[pl.ds(SKV_SHARD, SLIVER)],
                    send_sem=sliver_send_sems.at[i],
                    recv_sem=sliver_recv_sems.at[i],
                    device_id=(0,),
                    device_id_type=pl.DeviceIdType.MESH,
                )
                rd.start()
                sends.append(rd)
            for rd in sends:
                rd.wait_send()

        @pl.when(my == 0)
        def _():
            kbuf[pl.ds(0, SKV_SHARD)] = k_ref[...]
            vbuf[pl.ds(0, SKV_SHARD)] = v_ref[...]
            qbuf[...] = _mm(x_ref[...], wq_ref[...])

            sends = []
            for b in range(NBLK):
                ks = _KSTART[b]
                if ks + KBAND > SKV_SHARD:
                    for i, dst in enumerate((kbuf, vbuf)):
                        rd = pltpu.make_async_remote_copy(
                            src_ref=dst.at[pl.ds(SKV_SHARD, SLIVER)],
                            dst_ref=dst.at[pl.ds(SKV_SHARD, SLIVER)],
                            send_sem=sliver_send_sems.at[i],
                            recv_sem=sliver_recv_sems.at[i],
                            device_id=(1,),
                            device_id_type=pl.DeviceIdType.MESH,
                        )
                        rd.wait_recv()

                qi = b * QBLK + lax.broadcasted_iota(jnp.int32, (QBLK, KBAND), 0)
                kj = ks + lax.broadcasted_iota(jnp.int32, (QBLK, KBAND), 1)
                mask = jnp.abs(qi - kj) <= WIN

                for h in range(HQ):
                    q_h = qbuf[pl.ds(b * QBLK, QBLK), pl.ds(h * DH, DH)]
                    k_h = kbuf[pl.ds(ks, KBAND), h, :]
                    v_h = vbuf[pl.ds(ks, KBAND), h, :]
                    s = lax.dot_general(
                        q_h, k_h, (((1,), (1,)), ((), ())),
                        preferred_element_type=jnp.float32) * SCALE
                    s = jnp.where(mask, s, -1e30)
                    mx = jnp.max(s, axis=1, keepdims=True)
                    p = jnp.exp(s - mx)
                    p = jnp.where(mask, p, 0.0)
                    l = jnp.sum(p, axis=1, keepdims=True)
                    ctx_h = lax.dot_general(
                        p / l, v_h, (((1,), (0,)), ((), ())),
                        preferred_element_type=jnp.float32)
                    ctxbuf[:, pl.ds(h * DH, DH)] = ctx_h

                out_ref[pl.ds(b * QBLK, QBLK), :] = _mm(ctxbuf[...], wo_ref[...])

                for t, tgt in enumerate((1, 3)):
                    rd = pltpu.make_async_remote_copy(
                        src_ref=blk(out_ref, b),
                        dst_ref=blk(out_ref, b),
                        send_sem=send_sems.at[b, t],
                        recv_sem=recv_sems.at[b],
                        device_id=(tgt,),
                        device_id_type=pl.DeviceIdType.MESH,
                    )
                    rd.start()
                    sends.append(rd)
            for rd in sends:
                rd.wait_send()

        def recv_and_relay(relay_blocks):
            def go():
                relays = []
                for b in range(NBLK):
                    recv = pltpu.make_async_remote_copy(
                        src_ref=blk(out_ref, b),
                        dst_ref=blk(out_ref, b),
                        send_sem=send_sems.at[b, 0],
                        recv_sem=recv_sems.at[b],
                        device_id=(0,),
                        device_id_type=pl.DeviceIdType.MESH,
                    )
                    recv.wait_recv()
                    if b in relay_blocks:
                        rd = pltpu.make_async_remote_copy(
                            src_ref=blk(out_ref, b),
                            dst_ref=blk(out_ref, b),
                            send_sem=relay_sems.at[b],
                            recv_sem=recv_sems.at[b],
                            device_id=(2,),
                            device_id_type=pl.DeviceIdType.MESH,
                        )
                        rd.start()
                        relays.append(rd)
                for rd in relays:
                    rd.wait_send()
            return go

        pl.when(my == 1)(recv_and_relay((0, 1)))
        pl.when(my == 3)(recv_and_relay((2, 3)))

        @pl.when(my == 2)
        def _():
            for b in range(NBLK):
                recv = pltpu.make_async_remote_copy(
                    src_ref=blk(out_ref, b),
                    dst_ref=blk(out_ref, b),
                    send_sem=relay_sems.at[b],
                    recv_sem=recv_sems.at[b],
                    device_id=(0,),
                    device_id_type=pl.DeviceIdType.MESH,
                )
                recv.wait_recv()

    out = pl.pallas_call(
        body,
        out_shape=jax.ShapeDtypeStruct((SQ, D), jnp.float32),
        in_specs=[pl.BlockSpec(memory_space=pltpu.VMEM)] * 5,
        out_specs=pl.BlockSpec(memory_space=pltpu.VMEM),
        scratch_shapes=[
            pltpu.VMEM((SQ, D), jnp.float32),
            pltpu.VMEM((KV_USED, HQ, DH), jnp.float32),
            pltpu.VMEM((KV_USED, HQ, DH), jnp.float32),
            pltpu.VMEM((QBLK, D), jnp.float32),
            pltpu.SemaphoreType.DMA((2,)),
            pltpu.SemaphoreType.DMA((2,)),
            pltpu.SemaphoreType.DMA((NBLK, 2)),
            pltpu.SemaphoreType.DMA((NBLK,)),
            pltpu.SemaphoreType.DMA((NBLK,)),
        ],
        compiler_params=pltpu.CompilerParams(collective_id=0),
    )(x2, Wq, k3, v3, Wo)

    return out.reshape(1, SQ, D)


# baseline (device time: 82816 ns/iter reference)
import jax
import jax.numpy as jnp
from jax import lax
from jax.experimental import pallas as pl
from jax.experimental.pallas import tpu as pltpu

SQ = 1024
SKV_SHARD = 1024
HQ = 8
DH = 128
D = HQ * DH
WIN = 128
SCALE = 0.08838834764831843

QBLK = 256
NBLK = SQ // QBLK
KBAND = 512
SLIVER = 128
KV_USED = SKV_SHARD + SLIVER

_KSTART = (0, 128, 384, 640)


def _mm(a, b):
    return lax.dot_general(a, b, (((1,), (0,)), ((), ())),
                           preferred_element_type=jnp.float32)


def kernel(x, Wq, K_ext, V_ext, Wo):
    x2 = x.reshape(SQ, D)
    k3 = K_ext.reshape(SKV_SHARD, HQ, DH)
    v3 = V_ext.reshape(SKV_SHARD, HQ, DH)

    def body(x_ref, wq_ref, k_ref, v_ref, wo_ref, out_ref,
             qbuf, kbuf, vbuf, ctxbuf,
             sliver_send_sems, sliver_recv_sems,
             send_sems, recv_sems, relay_sems):
        my = lax.axis_index("i")

        def blk(ref, b):
            return ref.at[pl.ds(b * QBLK, QBLK), :]

        @pl.when(my == 1)
        def _():
            sends = []
            for i, (src, dst) in enumerate(((k_ref, kbuf), (v_ref, vbuf))):
                rd = pltpu.make_async_remote_copy(
                    src_ref=src.at[pl.ds(0, SLIVER)],
                    dst_ref=dst.at[pl.ds(SKV_SHARD, SLIVER)],
                    send_sem=sliver_send_sems.at[i],
                    recv_sem=sliver_recv_sems.at[i],
                    device_id=(0,),
                    device_id_type=pl.DeviceIdType.MESH,
                )
                rd.start()
                sends.append(rd)
            for rd in sends:
                rd.wait_send()

        @pl.when(my == 0)
        def _():
            kbuf[pl.ds(0, SKV_SHARD)] = k_ref[...]
            vbuf[pl.ds(0, SKV_SHARD)] = v_ref[...]
            qbuf[...] = _mm(x_ref[...], wq_ref[...])

            sends = []
            for b in range(NBLK):
                ks = _KSTART[b]
                if ks + KBAND > SKV_SHARD:
                    for i, dst in enumerate((kbuf, vbuf)):
                        rd = pltpu.make_async_remote_copy(
                            src_ref=dst.at[pl.ds(SKV_SHARD, SLIVER)],
                            dst_ref=dst.at[pl.ds(SKV_SHARD, SLIVER)],
                            send_sem=sliver_send_sems.at[i],
                            recv_sem=sliver_recv_sems.at[i],
                            device_id=(1,),
                            device_id_type=pl.DeviceIdType.MESH,
                        )
                        rd.wait_recv()

                qi = b * QBLK + lax.broadcasted_iota(jnp.int32, (QBLK, KBAND), 0)
                kj = ks + lax.broadcasted_iota(jnp.int32, (QBLK, KBAND), 1)
                mask = jnp.abs(qi - kj) <= WIN

                for h in range(HQ):
                    q_h = qbuf[pl.ds(b * QBLK, QBLK), pl.ds(h * DH, DH)]
                    k_h = kbuf[pl.ds(ks, KBAND), h, :]
                    v_h = vbuf[pl.ds(ks, KBAND), h, :]
                    s = lax.dot_general(
                        q_h, k_h, (((1,), (1,)), ((), ())),
                        preferred_element_type=jnp.float32) * SCALE
                    s = jnp.where(mask, s, -1e30)
                    mx = jnp.max(s, axis=1, keepdims=True)
                    p = jnp.exp(s - mx)
                    p = jnp.where(mask, p, 0.0)
                    l = jnp.sum(p, axis=1, keepdims=True)
                    ctx_h = lax.dot_general(
                        p / l, v_h, (((1,), (0,)), ((), ())),
                        preferred_element_type=jnp.float32)
                    ctxbuf[:, pl.ds(h * DH, DH)] = ctx_h

                out_ref[pl.ds(b * QBLK, QBLK), :] = _mm(ctxbuf[...], wo_ref[...])

                for t, tgt in enumerate((1, 3)):
                    rd = pltpu.make_async_remote_copy(
                        src_ref=blk(out_ref, b),
                        dst_ref=blk(out_ref, b),
                        send_sem=send_sems.at[b, t],
                        recv_sem=recv_sems.at[b],
                        device_id=(tgt,),
                        device_id_type=pl.DeviceIdType.MESH,
                    )
                    rd.start()
                    sends.append(rd)
            for rd in sends:
                rd.wait_send()

        def recv_and_relay(relay_blocks):
            def go():
                relays = []
                for b in range(NBLK):
                    recv = pltpu.make_async_remote_copy(
                        src_ref=blk(out_ref, b),
                        dst_ref=blk(out_ref, b),
                        send_sem=send_sems.at[b, 0],
                        recv_sem=recv_sems.at[b],
                        device_id=(0,),
                        device_id_type=pl.DeviceIdType.MESH,
                    )
                    recv.wait_recv()
                    if b in relay_blocks:
                        rd = pltpu.make_async_remote_copy(
                            src_ref=blk(out_ref, b),
                            dst_ref=blk(out_ref, b),
                            send_sem=relay_sems.at[b],
                            recv_sem=recv_sems.at[b],
                            device_id=(2,),
                            device_id_type=pl.DeviceIdType.MESH,
                        )
                        rd.start()
                        relays.append(rd)
                for rd in relays:
                    rd.wait_send()
            return go

        pl.when(my == 1)(recv_and_relay((0, 1)))
        pl.when(my == 3)(recv_and_relay((2, 3)))

        @pl.when(my == 2)
        def _():
            for b in range(NBLK):
                recv = pltpu.make_async_remote_copy(
                    src_ref=blk(out_ref, b),
                    dst_ref=blk(out_ref, b),
                    send_sem=relay_sems.at[b],
                    recv_sem=recv_sems.at[b],
                    device_id=(0,),
                    device_id_type=pl.DeviceIdType.MESH,
                )
                recv.wait_recv()

    out = pl.pallas_call(
        body,
        out_shape=jax.ShapeDtypeStruct((SQ, D), jnp.float32),
        in_specs=[pl.BlockSpec(memory_space=pltpu.VMEM)] * 5,
        out_specs=pl.BlockSpec(memory_space=pltpu.VMEM),
        scratch_shapes=[
            pltpu.VMEM((SQ, D), jnp.float32),
            pltpu.VMEM((KV_USED, HQ, DH), jnp.float32),
            pltpu.VMEM((KV_USED, HQ, DH), jnp.float32),
            pltpu.VMEM((QBLK, D), jnp.float32),
            pltpu.SemaphoreType.DMA((2,)),
            pltpu.SemaphoreType.DMA((2,)),
            pltpu.SemaphoreType.DMA((NBLK, 2)),
            pltpu.SemaphoreType.DMA((NBLK,)),
            pltpu.SemaphoreType.DMA((NBLK,)),
        ],
    )(x2, Wq, k3, v3, Wo)

    return out.reshape(1, SQ, D)


# device time: 63821 ns/iter; 1.2976x vs baseline; 1.2976x over previous
import jax
import jax.numpy as jnp
from jax import lax
from jax.experimental import pallas as pl
from jax.experimental.pallas import tpu as pltpu

SQ = 1024
SKV_SHARD = 1024
HQ = 8
DH = 128
D = HQ * DH
WIN = 128
SCALE = 0.08838834764831843

QBLK = 256
NBLK = SQ // QBLK
KBAND = 512
SLIVER = 128
KV_USED = SKV_SHARD + SLIVER

_KSTART = (0, 128, 384, 640)

BF16 = jnp.bfloat16


def _mm(a, b):
    return lax.dot_general(a, b, (((1,), (0,)), ((), ())),
                           preferred_element_type=jnp.float32)


def kernel(x, Wq, K_ext, V_ext, Wo):
    x2 = x.reshape(SQ, D)
    k3 = K_ext.reshape(SKV_SHARD, HQ, DH)
    v3 = V_ext.reshape(SKV_SHARD, HQ, DH)

    def body(x_ref, wq_ref, k_ref, v_ref, wo_ref, out_ref,
             qbuf, kbuf, vbuf, ctxbuf, obuf, kvsliv,
             sliver_send_sems, sliver_recv_sems,
             send_sems, recv_sems, relay_sems):
        my = lax.axis_index("i")

        def blk(ref, b):
            return ref.at[pl.ds(b * QBLK, QBLK), :]

        @pl.when(my == 1)
        def _():
            kvsliv[0] = k_ref[pl.ds(0, SLIVER)].astype(BF16)
            kvsliv[1] = v_ref[pl.ds(0, SLIVER)].astype(BF16)
            rd = pltpu.make_async_remote_copy(
                src_ref=kvsliv,
                dst_ref=kvsliv,
                send_sem=sliver_send_sems.at[0],
                recv_sem=sliver_recv_sems.at[0],
                device_id=(0,),
                device_id_type=pl.DeviceIdType.MESH,
            )
            rd.start()
            rd.wait_send()

        @pl.when(my == 0)
        def _():
            kbuf[pl.ds(0, SKV_SHARD)] = k_ref[...].astype(BF16)
            vbuf[pl.ds(0, SKV_SHARD)] = v_ref[...].astype(BF16)
            qbuf[...] = (_mm(x_ref[...].astype(BF16),
                             wq_ref[...].astype(BF16)) * SCALE).astype(BF16)
            wo16 = wo_ref[...].astype(BF16)

            sends = []
            for b in range(NBLK):
                ks = _KSTART[b]
                if ks + KBAND > SKV_SHARD:
                    rd = pltpu.make_async_remote_copy(
                        src_ref=kvsliv,
                        dst_ref=kvsliv,
                        send_sem=sliver_send_sems.at[0],
                        recv_sem=sliver_recv_sems.at[0],
                        device_id=(1,),
                        device_id_type=pl.DeviceIdType.MESH,
                    )
                    rd.wait_recv()
                    kbuf[pl.ds(SKV_SHARD, SLIVER)] = kvsliv[0]
                    vbuf[pl.ds(SKV_SHARD, SLIVER)] = kvsliv[1]

                qi = b * QBLK + lax.broadcasted_iota(jnp.int32, (QBLK, KBAND), 0)
                kj = ks + lax.broadcasted_iota(jnp.int32, (QBLK, KBAND), 1)
                mask = jnp.abs(qi - kj) <= WIN

                for h in range(HQ):
                    q_h = qbuf[pl.ds(b * QBLK, QBLK), pl.ds(h * DH, DH)]
                    k_h = kbuf[pl.ds(ks, KBAND), h, :]
                    v_h = vbuf[pl.ds(ks, KBAND), h, :]
                    s = lax.dot_general(
                        q_h, k_h, (((1,), (1,)), ((), ())),
                        preferred_element_type=jnp.float32)
                    s = jnp.where(mask, s, -1e30)
                    mx = jnp.max(s, axis=1, keepdims=True)
                    p = jnp.exp(s - mx)
                    p = jnp.where(mask, p, 0.0)
                    l = jnp.sum(p, axis=1, keepdims=True)
                    ctx_h = lax.dot_general(
                        (p / l).astype(BF16), v_h, (((1,), (0,)), ((), ())),
                        preferred_element_type=jnp.float32)
                    ctxbuf[:, pl.ds(h * DH, DH)] = ctx_h.astype(BF16)

                out_blk = _mm(ctxbuf[...], wo16)
                out_ref[pl.ds(b * QBLK, QBLK), :] = out_blk
                obuf[pl.ds(b * QBLK, QBLK), :] = out_blk.astype(BF16)

                for t, tgt in enumerate((1, 3)):
                    rd = pltpu.make_async_remote_copy(
                        src_ref=blk(obuf, b),
                        dst_ref=blk(obuf, b),
                        send_sem=send_sems.at[b, t],
                        recv_sem=recv_sems.at[b],
                        device_id=(tgt,),
                        device_id_type=pl.DeviceIdType.MESH,
                    )
                    rd.start()
                    sends.append(rd)
            for rd in sends:
                rd.wait_send()

        def recv_and_relay(relay_blocks):
            def go():
                relays = []
                for b in range(NBLK):
                    recv = pltpu.make_async_remote_copy(
                        src_ref=blk(obuf, b),
                        dst_ref=blk(obuf, b),
                        send_sem=send_sems.at[b, 0],
                        recv_sem=recv_sems.at[b],
                        device_id=(0,),
                        device_id_type=pl.DeviceIdType.MESH,
                    )
                    recv.wait_recv()
                    if b in relay_blocks:
                        rd = pltpu.make_async_remote_copy(
                            src_ref=blk(obuf, b),
                            dst_ref=blk(obuf, b),
                            send_sem=relay_sems.at[b],
                            recv_sem=recv_sems.at[b],
                            device_id=(2,),
                            device_id_type=pl.DeviceIdType.MESH,
                        )
                        rd.start()
                        relays.append(rd)
                    out_ref[pl.ds(b * QBLK, QBLK), :] = (
                        obuf[pl.ds(b * QBLK, QBLK), :].astype(jnp.float32))
                for rd in relays:
                    rd.wait_send()
            return go

        pl.when(my == 1)(recv_and_relay((0, 1)))
        pl.when(my == 3)(recv_and_relay((2, 3)))

        @pl.when(my == 2)
        def _():
            for b in range(NBLK):
                recv = pltpu.make_async_remote_copy(
                    src_ref=blk(obuf, b),
                    dst_ref=blk(obuf, b),
                    send_sem=relay_sems.at[b],
                    recv_sem=recv_sems.at[b],
                    device_id=(0,),
                    device_id_type=pl.DeviceIdType.MESH,
                )
                recv.wait_recv()
                out_ref[pl.ds(b * QBLK, QBLK), :] = (
                    obuf[pl.ds(b * QBLK, QBLK), :].astype(jnp.float32))

    out = pl.pallas_call(
        body,
        out_shape=jax.ShapeDtypeStruct((SQ, D), jnp.float32),
        in_specs=[pl.BlockSpec(memory_space=pltpu.VMEM)] * 5,
        out_specs=pl.BlockSpec(memory_space=pltpu.VMEM),
        scratch_shapes=[
            pltpu.VMEM((SQ, D), BF16),
            pltpu.VMEM((KV_USED, HQ, DH), BF16),
            pltpu.VMEM((KV_USED, HQ, DH), BF16),
            pltpu.VMEM((QBLK, D), BF16),
            pltpu.VMEM((SQ, D), BF16),
            pltpu.VMEM((2, SLIVER, HQ, DH), BF16),
            pltpu.SemaphoreType.DMA((1,)),
            pltpu.SemaphoreType.DMA((1,)),
            pltpu.SemaphoreType.DMA((NBLK, 2)),
            pltpu.SemaphoreType.DMA((NBLK,)),
            pltpu.SemaphoreType.DMA((NBLK,)),
        ],
    )(x2, Wq, k3, v3, Wo)

    return out.reshape(1, SQ, D)


# device time: 51563 ns/iter; 1.6061x vs baseline; 1.2377x over previous
import jax
import jax.numpy as jnp
from jax import lax
from jax.experimental import pallas as pl
from jax.experimental.pallas import tpu as pltpu

SQ = 1024
SKV_SHARD = 1024
HQ = 8
DH = 128
D = HQ * DH
WIN = 128
SCALE = 0.08838834764831843

QBLK = 256
NBLK = SQ // QBLK
KBAND = 512
SLIVER = 128
KV_USED = SKV_SHARD + SLIVER

_KSTART = (0, 128, 384, 640)

BF16 = jnp.bfloat16


def _mm(a, b):
    return lax.dot_general(a, b, (((1,), (0,)), ((), ())),
                           preferred_element_type=jnp.float32)


def kernel(x, Wq, K_ext, V_ext, Wo):
    x2 = x.reshape(SQ, D)
    k3 = K_ext.reshape(SKV_SHARD, D)
    v3 = V_ext.reshape(SKV_SHARD, D)

    def body(x_ref, wq_ref, k_ref, v_ref, wo_ref, out_ref,
             qbuf, kbuf, vbuf, ctxbuf, obuf, kvsliv,
             sliver_send_sems, sliver_recv_sems,
             send_sems, recv_sems, relay_sems):
        my = lax.axis_index("i")

        def blk(ref, b):
            return ref.at[pl.ds(b * QBLK, QBLK), :]

        @pl.when(my == 1)
        def _():
            kvsliv[0] = k_ref[pl.ds(0, SLIVER), :].astype(BF16)
            kvsliv[1] = v_ref[pl.ds(0, SLIVER), :].astype(BF16)
            rd = pltpu.make_async_remote_copy(
                src_ref=kvsliv,
                dst_ref=kvsliv,
                send_sem=sliver_send_sems.at[0],
                recv_sem=sliver_recv_sems.at[0],
                device_id=(0,),
                device_id_type=pl.DeviceIdType.MESH,
            )
            rd.start()
            rd.wait_send()

        @pl.when(my == 0)
        def _():
            kbuf[pl.ds(0, SKV_SHARD), :] = k_ref[...].astype(BF16)
            vbuf[pl.ds(0, SKV_SHARD), :] = v_ref[...].astype(BF16)
            qbuf[...] = (_mm(x_ref[...].astype(BF16),
                             wq_ref[...].astype(BF16)) * SCALE).astype(BF16)
            wo16 = wo_ref[...].astype(BF16)

            sends = []
            for b in range(NBLK):
                ks = _KSTART[b]
                if ks + KBAND > SKV_SHARD:
                    rd = pltpu.make_async_remote_copy(
                        src_ref=kvsliv,
                        dst_ref=kvsliv,
                        send_sem=sliver_send_sems.at[0],
                        recv_sem=sliver_recv_sems.at[0],
                        device_id=(1,),
                        device_id_type=pl.DeviceIdType.MESH,
                    )
                    rd.wait_recv()
                    kbuf[pl.ds(SKV_SHARD, SLIVER), :] = kvsliv[0]
                    vbuf[pl.ds(SKV_SHARD, SLIVER), :] = kvsliv[1]

                qi = b * QBLK + lax.broadcasted_iota(jnp.int32, (QBLK, KBAND), 0)
                kj = ks + lax.broadcasted_iota(jnp.int32, (QBLK, KBAND), 1)
                mask = jnp.abs(qi - kj) <= WIN

                for h in range(HQ):
                    q_h = qbuf[pl.ds(b * QBLK, QBLK), pl.ds(h * DH, DH)]
                    k_h = kbuf[pl.ds(ks, KBAND), pl.ds(h * DH, DH)]
                    v_h = vbuf[pl.ds(ks, KBAND), pl.ds(h * DH, DH)]
                    s = lax.dot_general(
                        q_h, k_h, (((1,), (1,)), ((), ())),
                        preferred_element_type=jnp.float32)
                    s = jnp.where(mask, s, -1e30)
                    mx = jnp.max(s, axis=1, keepdims=True)
                    p = jnp.exp(s - mx)
                    p = jnp.where(mask, p, 0.0)
                    l = jnp.sum(p, axis=1, keepdims=True)
                    ctx_h = lax.dot_general(
                        (p / l).astype(BF16), v_h, (((1,), (0,)), ((), ())),
                        preferred_element_type=jnp.float32)
                    ctxbuf[:, pl.ds(h * DH, DH)] = ctx_h.astype(BF16)

                out_blk = _mm(ctxbuf[...], wo16)
                out_ref[pl.ds(b * QBLK, QBLK), :] = out_blk
                obuf[pl.ds(b * QBLK, QBLK), :] = out_blk.astype(BF16)

                for t, tgt in enumerate((1, 3)):
                    rd = pltpu.make_async_remote_copy(
                        src_ref=blk(obuf, b),
                        dst_ref=blk(obuf, b),
                        send_sem=send_sems.at[b, t],
                        recv_sem=recv_sems.at[b],
                        device_id=(tgt,),
                        device_id_type=pl.DeviceIdType.MESH,
                    )
                    rd.start()
                    sends.append(rd)
            for rd in sends:
                rd.wait_send()

        def recv_and_relay(relay_blocks):
            def go():
                relays = []
                for b in range(NBLK):
                    recv = pltpu.make_async_remote_copy(
                        src_ref=blk(obuf, b),
                        dst_ref=blk(obuf, b),
                        send_sem=send_sems.at[b, 0],
                        recv_sem=recv_sems.at[b],
                        device_id=(0,),
                        device_id_type=pl.DeviceIdType.MESH,
                    )
                    recv.wait_recv()
                    if b in relay_blocks:
                        rd = pltpu.make_async_remote_copy(
                            src_ref=blk(obuf, b),
                            dst_ref=blk(obuf, b),
                            send_sem=relay_sems.at[b],
                            recv_sem=recv_sems.at[b],
                            device_id=(2,),
                            device_id_type=pl.DeviceIdType.MESH,
                        )
                        rd.start()
                        relays.append(rd)
                    out_ref[pl.ds(b * QBLK, QBLK), :] = (
                        obuf[pl.ds(b * QBLK, QBLK), :].astype(jnp.float32))
                for rd in relays:
                    rd.wait_send()
            return go

        pl.when(my == 1)(recv_and_relay((0, 1)))
        pl.when(my == 3)(recv_and_relay((2, 3)))

        @pl.when(my == 2)
        def _():
            for b in range(NBLK):
                recv = pltpu.make_async_remote_copy(
                    src_ref=blk(obuf, b),
                    dst_ref=blk(obuf, b),
                    send_sem=relay_sems.at[b],
                    recv_sem=recv_sems.at[b],
                    device_id=(0,),
                    device_id_type=pl.DeviceIdType.MESH,
                )
                recv.wait_recv()
                out_ref[pl.ds(b * QBLK, QBLK), :] = (
                    obuf[pl.ds(b * QBLK, QBLK), :].astype(jnp.float32))

    out = pl.pallas_call(
        body,
        out_shape=jax.ShapeDtypeStruct((SQ, D), jnp.float32),
        in_specs=[pl.BlockSpec(memory_space=pltpu.VMEM)] * 5,
        out_specs=pl.BlockSpec(memory_space=pltpu.VMEM),
        scratch_shapes=[
            pltpu.VMEM((SQ, D), BF16),
            pltpu.VMEM((KV_USED, D), BF16),
            pltpu.VMEM((KV_USED, D), BF16),
            pltpu.VMEM((QBLK, D), BF16),
            pltpu.VMEM((SQ, D), BF16),
            pltpu.VMEM((2, SLIVER, D), BF16),
            pltpu.SemaphoreType.DMA((1,)),
            pltpu.SemaphoreType.DMA((1,)),
            pltpu.SemaphoreType.DMA((NBLK, 2)),
            pltpu.SemaphoreType.DMA((NBLK,)),
            pltpu.SemaphoreType.DMA((NBLK,)),
        ],
    )(x2, Wq, k3, v3, Wo)

    return out.reshape(1, SQ, D)


# device time: 44805 ns/iter; 1.8484x vs baseline; 1.1508x over previous
import jax
import jax.numpy as jnp
from jax import lax
from jax.experimental import pallas as pl
from jax.experimental.pallas import tpu as pltpu

SQ = 1024
SKV_SHARD = 1024
HQ = 8
DH = 128
D = HQ * DH
WIN = 128
SCALE = 0.08838834764831843

QBLK = 128
NBLK = SQ // QBLK
KBAND = 384
SLIVER = 128
KV_USED = SKV_SHARD + SLIVER

_KSTART = tuple(min(max(128 * b - 128, 0), KV_USED - KBAND) for b in range(NBLK))

BF16 = jnp.bfloat16


def _mm(a, b):
    return lax.dot_general(a, b, (((1,), (0,)), ((), ())),
                           preferred_element_type=jnp.float32)


def kernel(x, Wq, K_ext, V_ext, Wo):
    x2 = x.reshape(SQ, D)
    k3 = K_ext.reshape(SKV_SHARD, D)
    v3 = V_ext.reshape(SKV_SHARD, D)

    def body(x_ref, wq_ref, k_ref, v_ref, wo_ref, out_ref,
             qbuf, kbuf, vbuf, ctxbuf, obuf, kvsliv,
             sliver_send_sems, sliver_recv_sems,
             send_sems, recv_sems, relay_sems):
        my = lax.axis_index("i")

        def blk(ref, b):
            return ref.at[pl.ds(b * QBLK, QBLK), :]

        barrier_sem = pltpu.get_barrier_semaphore()
        even = (my == 0) | (my == 2)
        p0 = jnp.where(even, 1, 0)
        p1 = jnp.where(even, 3, 2)
        for p in (p0, p1):
            pl.semaphore_signal(barrier_sem, inc=1, device_id=(p,),
                                device_id_type=pl.DeviceIdType.MESH)
        pl.semaphore_wait(barrier_sem, 2)

        @pl.when(my == 1)
        def _():
            kvsliv[0] = k_ref[pl.ds(0, SLIVER), :].astype(BF16)
            kvsliv[1] = v_ref[pl.ds(0, SLIVER), :].astype(BF16)
            rd = pltpu.make_async_remote_copy(
                src_ref=kvsliv,
                dst_ref=kvsliv,
                send_sem=sliver_send_sems.at[0],
                recv_sem=sliver_recv_sems.at[0],
                device_id=(0,),
                device_id_type=pl.DeviceIdType.MESH,
            )
            rd.start()
            rd.wait_send()

        @pl.when(my == 0)
        def _():
            kbuf[pl.ds(0, SKV_SHARD), :] = k_ref[...].astype(BF16)
            vbuf[pl.ds(0, SKV_SHARD), :] = v_ref[...].astype(BF16)
            qbuf[...] = (_mm(x_ref[...].astype(BF16),
                             wq_ref[...].astype(BF16)) * SCALE).astype(BF16)
            wo16 = wo_ref[...].astype(BF16)

            sends = []
            for b in range(NBLK):
                ks = _KSTART[b]
                if ks + KBAND > SKV_SHARD:
                    rd = pltpu.make_async_remote_copy(
                        src_ref=kvsliv,
                        dst_ref=kvsliv,
                        send_sem=sliver_send_sems.at[0],
                        recv_sem=sliver_recv_sems.at[0],
                        device_id=(1,),
                        device_id_type=pl.DeviceIdType.MESH,
                    )
                    rd.wait_recv()
                    kbuf[pl.ds(SKV_SHARD, SLIVER), :] = kvsliv[0]
                    vbuf[pl.ds(SKV_SHARD, SLIVER), :] = kvsliv[1]

                qi = b * QBLK + lax.broadcasted_iota(jnp.int32, (QBLK, KBAND), 0)
                kj = ks + lax.broadcasted_iota(jnp.int32, (QBLK, KBAND), 1)
                mask = jnp.abs(qi - kj) <= WIN

                for h in range(HQ):
                    q_h = qbuf[pl.ds(b * QBLK, QBLK), pl.ds(h * DH, DH)]
                    k_h = kbuf[pl.ds(ks, KBAND), pl.ds(h * DH, DH)]
                    v_h = vbuf[pl.ds(ks, KBAND), pl.ds(h * DH, DH)]
                    s = lax.dot_general(
                        q_h, k_h, (((1,), (1,)), ((), ())),
                        preferred_element_type=jnp.float32)
                    p = jnp.exp(jnp.where(mask, s, -1e30))
                    l = jnp.sum(p, axis=1, keepdims=True)
                    ctx_h = lax.dot_general(
                        (p / l).astype(BF16), v_h, (((1,), (0,)), ((), ())),
                        preferred_element_type=jnp.float32)
                    ctxbuf[:, pl.ds(h * DH, DH)] = ctx_h.astype(BF16)

                out_blk = _mm(ctxbuf[...], wo16)
                out_ref[pl.ds(b * QBLK, QBLK), :] = out_blk
                obuf[pl.ds(b * QBLK, QBLK), :] = out_blk.astype(BF16)

                for t, tgt in enumerate((1, 3)):
                    rd = pltpu.make_async_remote_copy(
                        src_ref=blk(obuf, b),
                        dst_ref=blk(obuf, b),
                        send_sem=send_sems.at[b, t],
                        recv_sem=recv_sems.at[b],
                        device_id=(tgt,),
                        device_id_type=pl.DeviceIdType.MESH,
                    )
                    rd.start()
                    sends.append(rd)
            for rd in sends:
                rd.wait_send()

        def recv_and_relay(relay_blocks):
            def go():
                relays = []
                for b in range(NBLK):
                    recv = pltpu.make_async_remote_copy(
                        src_ref=blk(obuf, b),
                        dst_ref=blk(obuf, b),
                        send_sem=send_sems.at[b, 0],
                        recv_sem=recv_sems.at[b],
                        device_id=(0,),
                        device_id_type=pl.DeviceIdType.MESH,
                    )
                    recv.wait_recv()
                    if b in relay_blocks:
                        rd = pltpu.make_async_remote_copy(
                            src_ref=blk(obuf, b),
                            dst_ref=blk(obuf, b),
                            send_sem=relay_sems.at[b],
                            recv_sem=recv_sems.at[b],
                            device_id=(2,),
                            device_id_type=pl.DeviceIdType.MESH,
                        )
                        rd.start()
                        relays.append(rd)
                    out_ref[pl.ds(b * QBLK, QBLK), :] = (
                        obuf[pl.ds(b * QBLK, QBLK), :].astype(jnp.float32))
                for rd in relays:
                    rd.wait_send()
            return go

        pl.when(my == 1)(recv_and_relay(tuple(range(NBLK // 2))))
        pl.when(my == 3)(recv_and_relay(tuple(range(NBLK // 2, NBLK))))

        @pl.when(my == 2)
        def _():
            for b in range(NBLK):
                recv = pltpu.make_async_remote_copy(
                    src_ref=blk(obuf, b),
                    dst_ref=blk(obuf, b),
                    send_sem=relay_sems.at[b],
                    recv_sem=recv_sems.at[b],
                    device_id=(0,),
                    device_id_type=pl.DeviceIdType.MESH,
                )
                recv.wait_recv()
                out_ref[pl.ds(b * QBLK, QBLK), :] = (
                    obuf[pl.ds(b * QBLK, QBLK), :].astype(jnp.float32))

    out = pl.pallas_call(
        body,
        out_shape=jax.ShapeDtypeStruct((SQ, D), jnp.float32),
        in_specs=[pl.BlockSpec(memory_space=pltpu.VMEM)] * 5,
        out_specs=pl.BlockSpec(memory_space=pltpu.VMEM),
        scratch_shapes=[
            pltpu.VMEM((SQ, D), BF16),
            pltpu.VMEM((KV_USED, D), BF16),
            pltpu.VMEM((KV_USED, D), BF16),
            pltpu.VMEM((QBLK, D), BF16),
            pltpu.VMEM((SQ, D), BF16),
            pltpu.VMEM((2, SLIVER, D), BF16),
            pltpu.SemaphoreType.DMA((1,)),
            pltpu.SemaphoreType.DMA((1,)),
            pltpu.SemaphoreType.DMA((NBLK, 2)),
            pltpu.SemaphoreType.DMA((NBLK,)),
            pltpu.SemaphoreType.DMA((NBLK,)),
        ],
        compiler_params=pltpu.CompilerParams(collective_id=0),
    )(x2, Wq, k3, v3, Wo)

    return out.reshape(1, SQ, D)


# device time: 39307 ns/iter; 2.1069x vs baseline; 1.1399x over previous
import os

import jax
import jax.numpy as jnp
from jax import lax
from jax.experimental import pallas as pl
from jax.experimental.pallas import tpu as pltpu

SQ = 1024
SKV_SHARD = 1024
HQ = 8
DH = 128
D = HQ * DH
WIN = 128
SCALE = 0.08838834764831843

QBLK = 128
NBLK = SQ // QBLK
KBAND = 384
SLIVER = 128
KV_USED = SKV_SHARD + SLIVER

_KSTART = tuple(min(max(128 * b - 128, 0), KV_USED - KBAND) for b in range(NBLK))

BF16 = jnp.bfloat16
ABLATE = os.environ.get("SCBAND_ABLATE", "")


def _mm(a, b):
    return lax.dot_general(a, b, (((1,), (0,)), ((), ())),
                           preferred_element_type=jnp.float32)


def kernel(x, Wq, K_ext, V_ext, Wo):
    x2 = x.reshape(SQ, D)
    k3 = K_ext.reshape(SKV_SHARD, D)
    v3 = V_ext.reshape(SKV_SHARD, D)

    def body(x_ref, wq_ref, k_ref, v_ref, wo_ref, out_ref,
             qbuf, kbuf, vbuf, ctxbuf, obuf, kvsliv,
             sliver_send_sems, sliver_recv_sems,
             send_sems, recv_sems, relay_sems):
        my = lax.axis_index("i")

        def blk(ref, b):
            return ref.at[pl.ds(b * QBLK, QBLK), :]

        if ABLATE == "nocomm":
            kbuf[pl.ds(0, SKV_SHARD), :] = k_ref[...].astype(BF16)
            vbuf[pl.ds(0, SKV_SHARD), :] = v_ref[...].astype(BF16)
            qbuf[...] = (_mm(x_ref[...].astype(BF16),
                             wq_ref[...].astype(BF16)) * SCALE).astype(BF16)
            wo16 = wo_ref[...].astype(BF16)
            for b in range(NBLK):
                ks = _KSTART[b]
                qi = b * QBLK + lax.broadcasted_iota(jnp.int32, (QBLK, KBAND), 0)
                kj = ks + lax.broadcasted_iota(jnp.int32, (QBLK, KBAND), 1)
                mask = jnp.abs(qi - kj) <= WIN
                for h in range(HQ):
                    q_h = qbuf[pl.ds(b * QBLK, QBLK), pl.ds(h * DH, DH)]
                    k_h = kbuf[pl.ds(ks, KBAND), pl.ds(h * DH, DH)]
                    v_h = vbuf[pl.ds(ks, KBAND), pl.ds(h * DH, DH)]
                    s = lax.dot_general(
                        q_h, k_h, (((1,), (1,)), ((), ())),
                        preferred_element_type=jnp.float32)
                    p = jnp.exp(jnp.where(mask, s, -1e30))
                    l = jnp.sum(p, axis=1, keepdims=True)
                    ctx_h = lax.dot_general(
                        (p / l).astype(BF16), v_h, (((1,), (0,)), ((), ())),
                        preferred_element_type=jnp.float32)
                    ctxbuf[:, pl.ds(h * DH, DH)] = ctx_h.astype(BF16)
                out_blk = _mm(ctxbuf[...], wo16)
                out_ref[pl.ds(b * QBLK, QBLK), :] = out_blk
                obuf[pl.ds(b * QBLK, QBLK), :] = out_blk.astype(BF16)
            return

        barrier_sem = pltpu.get_barrier_semaphore()
        even = (my == 0) | (my == 2)
        p0 = jnp.where(even, 1, 0)
        p1 = jnp.where(even, 3, 2)
        for p in (p0, p1):
            pl.semaphore_signal(barrier_sem, inc=1, device_id=(p,),
                                device_id_type=pl.DeviceIdType.MESH)
        pl.semaphore_wait(barrier_sem, 2)

        @pl.when(my == 1)
        def _():
            kvsliv[0] = k_ref[pl.ds(0, SLIVER), :].astype(BF16)
            kvsliv[1] = v_ref[pl.ds(0, SLIVER), :].astype(BF16)
            rd = pltpu.make_async_remote_copy(
                src_ref=kvsliv,
                dst_ref=kvsliv,
                send_sem=sliver_send_sems.at[0],
                recv_sem=sliver_recv_sems.at[0],
                device_id=(0,),
                device_id_type=pl.DeviceIdType.MESH,
            )
            rd.start()
            rd.wait_send()

        @pl.when(my == 0)
        def _():
            if ABLATE == "nocompute":
                sends = []
                for b in range(NBLK):
                    for t, tgt in enumerate((1, 3)):
                        rd = pltpu.make_async_remote_copy(
                            src_ref=blk(obuf, b), dst_ref=blk(obuf, b),
                            send_sem=send_sems.at[b, t],
                            recv_sem=recv_sems.at[b],
                            device_id=(tgt,),
                            device_id_type=pl.DeviceIdType.MESH)
                        rd.start()
                        sends.append(rd)
                for rd in sends:
                    rd.wait_send()
                rd = pltpu.make_async_remote_copy(
                    src_ref=kvsliv, dst_ref=kvsliv,
                    send_sem=sliver_send_sems.at[0],
                    recv_sem=sliver_recv_sems.at[0],
                    device_id=(1,), device_id_type=pl.DeviceIdType.MESH)
                rd.wait_recv()
                return
            kbuf[pl.ds(0, SKV_SHARD), :] = k_ref[...].astype(BF16)
            vbuf[pl.ds(0, SKV_SHARD), :] = v_ref[...].astype(BF16)
            qbuf[...] = (_mm(x_ref[...].astype(BF16),
                             wq_ref[...].astype(BF16)) * SCALE).astype(BF16)
            wo16 = wo_ref[...].astype(BF16)

            sends = []
            for b in range(NBLK):
                ks = _KSTART[b]
                if ks + KBAND > SKV_SHARD:
                    rd = pltpu.make_async_remote_copy(
                        src_ref=kvsliv,
                        dst_ref=kvsliv,
                        send_sem=sliver_send_sems.at[0],
                        recv_sem=sliver_recv_sems.at[0],
                        device_id=(1,),
                        device_id_type=pl.DeviceIdType.MESH,
                    )
                    rd.wait_recv()
                    kbuf[pl.ds(SKV_SHARD, SLIVER), :] = kvsliv[0]
                    vbuf[pl.ds(SKV_SHARD, SLIVER), :] = kvsliv[1]

                qi = b * QBLK + lax.broadcasted_iota(jnp.int32, (QBLK, KBAND), 0)
                kj = ks + lax.broadcasted_iota(jnp.int32, (QBLK, KBAND), 1)
                mask = jnp.abs(qi - kj) <= WIN

                for h in range(HQ):
                    q_h = qbuf[pl.ds(b * QBLK, QBLK), pl.ds(h * DH, DH)]
                    k_h = kbuf[pl.ds(ks, KBAND), pl.ds(h * DH, DH)]
                    v_h = vbuf[pl.ds(ks, KBAND), pl.ds(h * DH, DH)]
                    s = lax.dot_general(
                        q_h, k_h, (((1,), (1,)), ((), ())),
                        preferred_element_type=jnp.float32)
                    p = jnp.exp(jnp.where(mask, s, -1e30))
                    l = jnp.sum(p, axis=1, keepdims=True)
                    ctx_h = lax.dot_general(
                        (p / l).astype(BF16), v_h, (((1,), (0,)), ((), ())),
                        preferred_element_type=jnp.float32)
                    ctxbuf[:, pl.ds(h * DH, DH)] = ctx_h.astype(BF16)

                out_blk = _mm(ctxbuf[...], wo16)
                out_ref[pl.ds(b * QBLK, QBLK), :] = out_blk
                obuf[pl.ds(b * QBLK, QBLK), :] = out_blk.astype(BF16)

                for t, tgt in enumerate((1, 3)):
                    rd = pltpu.make_async_remote_copy(
                        src_ref=blk(obuf, b),
                        dst_ref=blk(obuf, b),
                        send_sem=send_sems.at[b, t],
                        recv_sem=recv_sems.at[b],
                        device_id=(tgt,),
                        device_id_type=pl.DeviceIdType.MESH,
                    )
                    rd.start()
                    sends.append(rd)
            for rd in sends:
                rd.wait_send()

        def recv_and_relay(relay_blocks):
            def go():
                relays = []
                for b in range(NBLK):
                    recv = pltpu.make_async_remote_copy(
                        src_ref=blk(obuf, b),
                        dst_ref=blk(obuf, b),
                        send_sem=send_sems.at[b, 0],
                        recv_sem=recv_sems.at[b],
                        device_id=(0,),
                        device_id_type=pl.DeviceIdType.MESH,
                    )
                    recv.wait_recv()
                    if b in relay_blocks:
                        rd = pltpu.make_async_remote_copy(
                            src_ref=blk(obuf, b),
                            dst_ref=blk(obuf, b),
                            send_sem=relay_sems.at[b],
                            recv_sem=recv_sems.at[b],
                            device_id=(2,),
                            device_id_type=pl.DeviceIdType.MESH,
                        )
                        rd.start()
                        relays.append(rd)
                    out_ref[pl.ds(b * QBLK, QBLK), :] = (
                        obuf[pl.ds(b * QBLK, QBLK), :].astype(jnp.float32))
                for rd in relays:
                    rd.wait_send()
            return go

        pl.when(my == 1)(recv_and_relay(tuple(range(NBLK // 2))))
        pl.when(my == 3)(recv_and_relay(tuple(range(NBLK // 2, NBLK))))

        @pl.when(my == 2)
        def _():
            for b in range(NBLK):
                recv = pltpu.make_async_remote_copy(
                    src_ref=blk(obuf, b),
                    dst_ref=blk(obuf, b),
                    send_sem=relay_sems.at[b],
                    recv_sem=recv_sems.at[b],
                    device_id=(0,),
                    device_id_type=pl.DeviceIdType.MESH,
                )
                recv.wait_recv()
                out_ref[pl.ds(b * QBLK, QBLK), :] = (
                    obuf[pl.ds(b * QBLK, QBLK), :].astype(jnp.float32))

    out = pl.pallas_call(
        body,
        out_shape=jax.ShapeDtypeStruct((SQ, D), jnp.float32),
        in_specs=[pl.BlockSpec(memory_space=pltpu.VMEM)] * 5,
        out_specs=pl.BlockSpec(memory_space=pltpu.VMEM),
        scratch_shapes=[
            pltpu.VMEM((SQ, D), BF16),
            pltpu.VMEM((KV_USED, D), BF16),
            pltpu.VMEM((KV_USED, D), BF16),
            pltpu.VMEM((QBLK, D), BF16),
            pltpu.VMEM((SQ, D), BF16),
            pltpu.VMEM((2, SLIVER, D), BF16),
            pltpu.SemaphoreType.DMA((1,)),
            pltpu.SemaphoreType.DMA((1,)),
            pltpu.SemaphoreType.DMA((NBLK, 2)),
            pltpu.SemaphoreType.DMA((NBLK,)),
            pltpu.SemaphoreType.DMA((NBLK,)),
        ],
        compiler_params=pltpu.CompilerParams(collective_id=0),
    )(x2, Wq, k3, v3, Wo)

    return out.reshape(1, SQ, D)


# device time: 13263 ns/iter; 6.2441x vs baseline; 2.9637x over previous
import os

import jax
import jax.numpy as jnp
from jax import lax
from jax.experimental import pallas as pl
from jax.experimental.pallas import tpu as pltpu

SQ = 1024
SKV_SHARD = 1024
HQ = 8
DH = 128
D = HQ * DH
WIN = 128
SCALE = 0.08838834764831843

QBLK = 128
NBLK = SQ // QBLK
KBAND = 384
SLIVER = 128
KV_USED = SKV_SHARD + SLIVER

_KSTART = tuple(min(max(128 * b - 128, 0), KV_USED - KBAND) for b in range(NBLK))

BF16 = jnp.bfloat16
ABLATE = os.environ.get("SCBAND_ABLATE", "")


def _mm(a, b):
    return lax.dot_general(a, b, (((1,), (0,)), ((), ())),
                           preferred_element_type=jnp.float32)


def kernel(x, Wq, K_ext, V_ext, Wo):
    x2 = x.reshape(SQ, D)
    k3 = K_ext.reshape(SKV_SHARD, D)
    v3 = V_ext.reshape(SKV_SHARD, D)

    def body(x_ref, wq_ref, k_ref, v_ref, wo_ref, out_ref,
             qbuf, kbuf, vbuf, ctxbuf, obuf, kvsliv,
             sliver_send_sems, sliver_recv_sems,
             send_sems, recv_sems, relay_sems):
        my = lax.axis_index("i")

        def blk(ref, b):
            return ref.at[pl.ds(b * QBLK, QBLK), :]

        if ABLATE == "barrieronly":
            barrier_sem = pltpu.get_barrier_semaphore()
            even = (my == 0) | (my == 2)
            p0 = jnp.where(even, 1, 0)
            p1 = jnp.where(even, 3, 2)
            for p in (p0, p1):
                pl.semaphore_signal(barrier_sem, inc=1, device_id=(p,),
                                    device_id_type=pl.DeviceIdType.MESH)
            pl.semaphore_wait(barrier_sem, 2)
            out_ref[...] = x_ref[...]
            return

        if ABLATE == "nocomm":
            kbuf[pl.ds(0, SKV_SHARD), :] = k_ref[...].astype(BF16)
            vbuf[pl.ds(0, SKV_SHARD), :] = v_ref[...].astype(BF16)
            qbuf[...] = (_mm(x_ref[...].astype(BF16),
                             wq_ref[...].astype(BF16)) * SCALE).astype(BF16)
            wo16 = wo_ref[...].astype(BF16)
            for b in range(NBLK):
                ks = _KSTART[b]
                qi = b * QBLK + lax.broadcasted_iota(jnp.int32, (QBLK, KBAND), 0)
                kj = ks + lax.broadcasted_iota(jnp.int32, (QBLK, KBAND), 1)
                mask = jnp.abs(qi - kj) <= WIN
                for h in range(HQ):
                    q_h = qbuf[pl.ds(b * QBLK, QBLK), pl.ds(h * DH, DH)]
                    k_h = kbuf[pl.ds(ks, KBAND), pl.ds(h * DH, DH)]
                    v_h = vbuf[pl.ds(ks, KBAND), pl.ds(h * DH, DH)]
                    s = lax.dot_general(
                        q_h, k_h, (((1,), (1,)), ((), ())),
                        preferred_element_type=jnp.float32)
                    p = jnp.exp(jnp.where(mask, s, -1e30))
                    l = jnp.sum(p, axis=1, keepdims=True)
                    ctx_h = lax.dot_general(
                        (p / l).astype(BF16), v_h, (((1,), (0,)), ((), ())),
                        preferred_element_type=jnp.float32)
                    ctxbuf[:, pl.ds(h * DH, DH)] = ctx_h.astype(BF16)
                out_blk = _mm(ctxbuf[...], wo16)
                out_ref[pl.ds(b * QBLK, QBLK), :] = out_blk
                obuf[pl.ds(b * QBLK, QBLK), :] = out_blk.astype(BF16)
            return

        barrier_sem = pltpu.get_barrier_semaphore()
        even = (my == 0) | (my == 2)
        p0 = jnp.where(even, 1, 0)
        p1 = jnp.where(even, 3, 2)
        for p in (p0, p1):
            pl.semaphore_signal(barrier_sem, inc=1, device_id=(p,),
                                device_id_type=pl.DeviceIdType.MESH)
        pl.semaphore_wait(barrier_sem, 2)

        @pl.when(my == 1)
        def _():
            kvsliv[0] = k_ref[pl.ds(0, SLIVER), :].astype(BF16)
            kvsliv[1] = v_ref[pl.ds(0, SLIVER), :].astype(BF16)
            rd = pltpu.make_async_remote_copy(
                src_ref=kvsliv,
                dst_ref=kvsliv,
                send_sem=sliver_send_sems.at[0],
                recv_sem=sliver_recv_sems.at[0],
                device_id=(0,),
                device_id_type=pl.DeviceIdType.MESH,
            )
            rd.start()
            rd.wait_send()

        @pl.when(my == 0)
        def _():
            if ABLATE == "nocompute":
                sends = []
                for b in range(NBLK):
                    for t, tgt in enumerate((1, 3)):
                        rd = pltpu.make_async_remote_copy(
                            src_ref=blk(obuf, b), dst_ref=blk(obuf, b),
                            send_sem=send_sems.at[b, t],
                            recv_sem=recv_sems.at[b],
                            device_id=(tgt,),
                            device_id_type=pl.DeviceIdType.MESH)
                        rd.start()
                        sends.append(rd)
                for rd in sends:
                    rd.wait_send()
                rd = pltpu.make_async_remote_copy(
                    src_ref=kvsliv, dst_ref=kvsliv,
                    send_sem=sliver_send_sems.at[0],
                    recv_sem=sliver_recv_sems.at[0],
                    device_id=(1,), device_id_type=pl.DeviceIdType.MESH)
                rd.wait_recv()
                return
            kbuf[pl.ds(0, SKV_SHARD), :] = k_ref[...].astype(BF16)
            vbuf[pl.ds(0, SKV_SHARD), :] = v_ref[...].astype(BF16)
            qbuf[...] = (_mm(x_ref[...].astype(BF16),
                             wq_ref[...].astype(BF16)) * SCALE).astype(BF16)
            wo16 = wo_ref[...].astype(BF16)

            sends = []
            for b in range(NBLK):
                ks = _KSTART[b]
                if ks + KBAND > SKV_SHARD:
                    rd = pltpu.make_async_remote_copy(
                        src_ref=kvsliv,
                        dst_ref=kvsliv,
                        send_sem=sliver_send_sems.at[0],
                        recv_sem=sliver_recv_sems.at[0],
                        device_id=(1,),
                        device_id_type=pl.DeviceIdType.MESH,
                    )
                    rd.wait_recv()
                    kbuf[pl.ds(SKV_SHARD, SLIVER), :] = kvsliv[0]
                    vbuf[pl.ds(SKV_SHARD, SLIVER), :] = kvsliv[1]

                qi = b * QBLK + lax.broadcasted_iota(jnp.int32, (QBLK, KBAND), 0)
                kj = ks + lax.broadcasted_iota(jnp.int32, (QBLK, KBAND), 1)
                mask = jnp.abs(qi - kj) <= WIN

                for h in range(HQ):
                    q_h = qbuf[pl.ds(b * QBLK, QBLK), pl.ds(h * DH, DH)]
                    k_h = kbuf[pl.ds(ks, KBAND), pl.ds(h * DH, DH)]
                    v_h = vbuf[pl.ds(ks, KBAND), pl.ds(h * DH, DH)]
                    s = lax.dot_general(
                        q_h, k_h, (((1,), (1,)), ((), ())),
                        preferred_element_type=jnp.float32)
                    p = jnp.exp(jnp.where(mask, s, -1e30))
                    l = jnp.sum(p, axis=1, keepdims=True)
                    ctx_h = lax.dot_general(
                        (p / l).astype(BF16), v_h, (((1,), (0,)), ((), ())),
                        preferred_element_type=jnp.float32)
                    ctxbuf[:, pl.ds(h * DH, DH)] = ctx_h.astype(BF16)

                out_blk = _mm(ctxbuf[...], wo16)
                out_ref[pl.ds(b * QBLK, QBLK), :] = out_blk
                obuf[pl.ds(b * QBLK, QBLK), :] = out_blk.astype(BF16)

                for t, tgt in enumerate((1, 3)):
                    rd = pltpu.make_async_remote_copy(
                        src_ref=blk(obuf, b),
                        dst_ref=blk(obuf, b),
                        send_sem=send_sems.at[b, t],
                        recv_sem=recv_sems.at[b],
                        device_id=(tgt,),
                        device_id_type=pl.DeviceIdType.MESH,
                    )
                    rd.start()
                    sends.append(rd)
            for rd in sends:
                rd.wait_send()

        def recv_and_relay(relay_blocks):
            def go():
                relays = []
                for b in range(NBLK):
                    recv = pltpu.make_async_remote_copy(
                        src_ref=blk(obuf, b),
                        dst_ref=blk(obuf, b),
                        send_sem=send_sems.at[b, 0],
                        recv_sem=recv_sems.at[b],
                        device_id=(0,),
                        device_id_type=pl.DeviceIdType.MESH,
                    )
                    recv.wait_recv()
                    if b in relay_blocks:
                        rd = pltpu.make_async_remote_copy(
                            src_ref=blk(obuf, b),
                            dst_ref=blk(obuf, b),
                            send_sem=relay_sems.at[b],
                            recv_sem=recv_sems.at[b],
                            device_id=(2,),
                            device_id_type=pl.DeviceIdType.MESH,
                        )
                        rd.start()
                        relays.append(rd)
                    out_ref[pl.ds(b * QBLK, QBLK), :] = (
                        obuf[pl.ds(b * QBLK, QBLK), :].astype(jnp.float32))
                for rd in relays:
                    rd.wait_send()
            return go

        pl.when(my == 1)(recv_and_relay(tuple(range(NBLK // 2))))
        pl.when(my == 3)(recv_and_relay(tuple(range(NBLK // 2, NBLK))))

        @pl.when(my == 2)
        def _():
            for b in range(NBLK):
                recv = pltpu.make_async_remote_copy(
                    src_ref=blk(obuf, b),
                    dst_ref=blk(obuf, b),
                    send_sem=relay_sems.at[b],
                    recv_sem=recv_sems.at[b],
                    device_id=(0,),
                    device_id_type=pl.DeviceIdType.MESH,
                )
                recv.wait_recv()
                out_ref[pl.ds(b * QBLK, QBLK), :] = (
                    obuf[pl.ds(b * QBLK, QBLK), :].astype(jnp.float32))

    out = pl.pallas_call(
        body,
        out_shape=jax.ShapeDtypeStruct((SQ, D), jnp.float32),
        in_specs=[pl.BlockSpec(memory_space=pltpu.VMEM)] * 5,
        out_specs=pl.BlockSpec(memory_space=pltpu.VMEM),
        scratch_shapes=[
            pltpu.VMEM((SQ, D), BF16),
            pltpu.VMEM((KV_USED, D), BF16),
            pltpu.VMEM((KV_USED, D), BF16),
            pltpu.VMEM((QBLK, D), BF16),
            pltpu.VMEM((SQ, D), BF16),
            pltpu.VMEM((2, SLIVER, D), BF16),
            pltpu.SemaphoreType.DMA((1,)),
            pltpu.SemaphoreType.DMA((1,)),
            pltpu.SemaphoreType.DMA((NBLK, 2)),
            pltpu.SemaphoreType.DMA((NBLK,)),
            pltpu.SemaphoreType.DMA((NBLK,)),
        ],
        compiler_params=pltpu.CompilerParams(collective_id=0),
    )(x2, Wq, k3, v3, Wo)

    return out.reshape(1, SQ, D)
